# Initial kernel scaffold; baseline (speedup 1.0000x reference)
import jax
import jax.numpy as jnp
from jax.experimental import pallas as pl


def kernel(embeddings):
    raise NotImplementedError("write your pallas kernel here")


# speedup vs baseline: 5.7817x; 5.7817x over previous
import jax
import jax.numpy as jnp
from jax.experimental import pallas as pl
from jax.experimental.pallas import tpu as pltpu

_N = 10000
_D = 128
_K = 20
_RB = 400
_NBLK = _N // _RB
_NEG = -3.0


def _prep_body(x_ref, o_ref):
    x = x_ref[...]
    s = jnp.sum(x * x, axis=1, keepdims=True)
    y = x / jnp.sqrt(jnp.maximum(s, 1e-12))
    o_ref[...] = y.astype(jnp.bfloat16)


def _topk_body(rows_ref, all_ref, vals_ref, idxs_ref):
    i = pl.program_id(0)
    a = rows_ref[...]
    b = all_ref[...]
    sim = jax.lax.dot_general(
        a, b, (((1,), (1,)), ((), ())), preferred_element_type=jnp.float32
    )
    col = jax.lax.broadcasted_iota(jnp.int32, (_RB, _N), 1)
    row = jax.lax.broadcasted_iota(jnp.int32, (_RB, _N), 0) + i * _RB
    s = jnp.where(col == row, -2.0, sim)
    vals = []
    idxs = []
    for _ in range(_K):
        m = jnp.max(s, axis=1, keepdims=True)
        idx = jnp.min(jnp.where(s == m, col, _N), axis=1, keepdims=True)
        vals.append(m)
        idxs.append(idx)
        s = jnp.where(col == idx, _NEG, s)
    vals_ref[...] = jnp.concatenate(vals, axis=1)
    idxs_ref[...] = jnp.concatenate(idxs, axis=1)


def kernel(embeddings):
    norm_bf16 = pl.pallas_call(
        _prep_body,
        out_shape=jax.ShapeDtypeStruct((_N, _D), jnp.bfloat16),
    )(embeddings)

    vals, idxs = pl.pallas_call(
        _topk_body,
        grid=(_NBLK,),
        in_specs=[
            pl.BlockSpec((_RB, _D), lambda i: (i, 0)),
            pl.BlockSpec((_N, _D), lambda i: (0, 0)),
        ],
        out_specs=[
            pl.BlockSpec((_RB, _K), lambda i: (i, 0)),
            pl.BlockSpec((_RB, _K), lambda i: (i, 0)),
        ],
        out_shape=[
            jax.ShapeDtypeStruct((_N, _K), jnp.float32),
            jax.ShapeDtypeStruct((_N, _K), jnp.int32),
        ],
        compiler_params=pltpu.CompilerParams(
            dimension_semantics=("arbitrary",),
        ),
    )(norm_bf16, norm_bf16)
    return vals, idxs
